# baseline (speedup 1.0000x reference)
import functools

import jax
import jax.numpy as jnp
from jax import lax
from jax.experimental import pallas as pl
from jax.experimental.pallas import tpu as pltpu
from jax.experimental.pallas import tpu_sc as plsc
from jax.experimental.compute_on import compute_on


def _sc_gather(table, idx):
    with compute_on("tpu_sparsecore"):
        return table[idx]


def _leaky_relu(x, slope=0.2):
    return jnp.where(x >= 0, x, slope * x)


def _make_sc_row_gather(V, D, B_pad, CH, NC, NW):
    b_per_w = B_pad // NW
    nch = b_per_w // CH
    mesh = plsc.VectorSubcoreMesh(core_axis_name="c", subcore_axis_name="s")

    @functools.partial(
        pl.kernel,
        mesh=mesh,
        out_type=jax.ShapeDtypeStruct((B_pad, D), jnp.float32),
        scratch_types=[
            pltpu.VMEM((CH,), jnp.int32),
            pltpu.VMEM((CH, D), jnp.float32),
            pltpu.SemaphoreType.DMA,
        ],
    )
    def k(table_hbm, idx_hbm, out_hbm, idx_v, rows_v, sem):
        wid = lax.axis_index("s") * NC + lax.axis_index("c")
        base = wid * b_per_w

        @pl.loop(0, nch)
        def _(i):
            off = base + i * CH
            pltpu.sync_copy(idx_hbm.at[pl.ds(off, CH)], idx_v)
            pltpu.async_copy(table_hbm.at[idx_v], rows_v, sem).wait()
            pltpu.sync_copy(rows_v, out_hbm.at[pl.ds(off, CH)])

    return k


def _pallas_gather_rows(table, idx, CH=512):
    info = plsc.get_sparse_core_info()
    NC, NW = info.num_cores, info.num_cores * info.num_subcores
    B = idx.shape[0]
    step = NW * CH
    B_pad = ((B + step - 1) // step) * step
    if B_pad != B:
        idx = jnp.concatenate(
            [idx, jnp.zeros((B_pad - B,), dtype=idx.dtype)])
    k = _make_sc_row_gather(table.shape[0], table.shape[1], B_pad, CH, NC, NW)
    out = k(table, idx)
    return out[:B] if B_pad != B else out


def _gat_conv(x, src, dst, W, a_src, a_dst, b, n, perm, sdst, row_gather):
    h = x @ W
    alpha = (_sc_gather((h * a_src).sum(axis=-1), src)
             + _sc_gather((h * a_dst).sum(axis=-1), dst))
    alpha = _leaky_relu(alpha, 0.2)
    amax = jax.ops.segment_max(alpha[perm], sdst, num_segments=n,
                               indices_are_sorted=True)
    amax = jnp.where(jnp.isfinite(amax), amax, 0.0)
    e = jnp.exp(alpha - _sc_gather(amax, dst))
    denom = jax.ops.segment_sum(e[perm], sdst, num_segments=n,
                                indices_are_sorted=True)
    coef = e / (_sc_gather(denom, dst) + 1e-16)
    hsrc = row_gather(h, src)
    out = jax.ops.segment_sum(coef[:, None] * hsrc, dst, num_segments=n)
    return out + b


def kernel(x, edge_index, W1, a_src1, a_dst1, b1, W2, a_src2, a_dst2, b2):
    n = x.shape[0]
    loops = jnp.arange(n, dtype=edge_index.dtype)
    src = jnp.concatenate([edge_index[0], loops])
    dst = jnp.concatenate([edge_index[1], loops])
    iota = jnp.arange(dst.shape[0], dtype=jnp.int32)
    sdst, perm = jax.lax.sort((dst.astype(jnp.int32), iota), num_keys=1,
                              is_stable=False)
    x1 = jax.nn.relu(_gat_conv(x, src, dst, W1, a_src1, a_dst1, b1, n, perm,
                               sdst, _pallas_gather_rows))
    x2 = _gat_conv(x1, src, dst, W2, a_src2, a_dst2, b2, n, perm, sdst,
                   _sc_gather)
    value = (_sc_gather(x2, edge_index[0]) * _sc_gather(x2, edge_index[1])).sum(axis=1)
    E = value.shape[0]
    k_homo = int(E * 0.95)
    k_het = int(E * 0.05)
    _, topk_homo = jax.lax.top_k(value, k_homo)
    _, topk_hetero = jax.lax.top_k(-value, k_het)
    ei_t = edge_index.T
    homo = _sc_gather(ei_t, topk_homo).T
    het = _sc_gather(ei_t, topk_hetero).T
    return homo, het


# speedup vs baseline: 1.0471x; 1.0471x over previous
import functools

import jax
import jax.numpy as jnp
from jax import lax
from jax.experimental import pallas as pl
from jax.experimental.pallas import tpu as pltpu
from jax.experimental.pallas import tpu_sc as plsc
from jax.experimental.compute_on import compute_on


def _sc_gather(table, idx):
    with compute_on("tpu_sparsecore"):
        return table[idx]


def _leaky_relu(x, slope=0.2):
    return jnp.where(x >= 0, x, slope * x)


def _make_sc_row_gather(V, D, B_pad, CH, NC, NW):
    b_per_w = B_pad // NW
    nch = b_per_w // CH
    mesh = plsc.VectorSubcoreMesh(core_axis_name="c", subcore_axis_name="s")

    @functools.partial(
        pl.kernel,
        mesh=mesh,
        out_type=jax.ShapeDtypeStruct((B_pad, D), jnp.float32),
        scratch_types=[
            pltpu.VMEM((CH,), jnp.int32),
            pltpu.VMEM((CH, D), jnp.float32),
            pltpu.SemaphoreType.DMA,
        ],
    )
    def k(table_hbm, idx_hbm, out_hbm, idx_v, rows_v, sem):
        wid = lax.axis_index("s") * NC + lax.axis_index("c")
        base = wid * b_per_w

        @pl.loop(0, nch)
        def _(i):
            off = base + i * CH
            pltpu.sync_copy(idx_hbm.at[pl.ds(off, CH)], idx_v)
            pltpu.async_copy(table_hbm.at[idx_v], rows_v, sem).wait()
            pltpu.sync_copy(rows_v, out_hbm.at[pl.ds(off, CH)])

    return k


def _pallas_gather_rows(table, idx, CH=256):
    info = plsc.get_sparse_core_info()
    NC, NW = info.num_cores, info.num_cores * info.num_subcores
    B = idx.shape[0]
    step = NW * CH
    B_pad = ((B + step - 1) // step) * step
    if B_pad != B:
        idx = jnp.concatenate(
            [idx, jnp.zeros((B_pad - B,), dtype=idx.dtype)])
    k = _make_sc_row_gather(table.shape[0], table.shape[1], B_pad, CH, NC, NW)
    out = k(table, idx)
    return out[:B] if B_pad != B else out


def _gat_conv(x, src, dst, W, a_src, a_dst, b, n, perm, sdst, row_gather):
    h = x @ W
    alpha = (_sc_gather((h * a_src).sum(axis=-1), src)
             + _sc_gather((h * a_dst).sum(axis=-1), dst))
    alpha = _leaky_relu(alpha, 0.2)
    amax = jax.ops.segment_max(alpha[perm], sdst, num_segments=n,
                               indices_are_sorted=True)
    amax = jnp.where(jnp.isfinite(amax), amax, 0.0)
    e = jnp.exp(alpha - _sc_gather(amax, dst))
    denom = jax.ops.segment_sum(e[perm], sdst, num_segments=n,
                                indices_are_sorted=True)
    coef = e / (_sc_gather(denom, dst) + 1e-16)
    hsrc = row_gather(h, src)
    out = jax.ops.segment_sum(coef[:, None] * hsrc, dst, num_segments=n)
    return out + b


def kernel(x, edge_index, W1, a_src1, a_dst1, b1, W2, a_src2, a_dst2, b2):
    n = x.shape[0]
    loops = jnp.arange(n, dtype=edge_index.dtype)
    src = jnp.concatenate([edge_index[0], loops])
    dst = jnp.concatenate([edge_index[1], loops])
    iota = jnp.arange(dst.shape[0], dtype=jnp.int32)
    sdst, perm = jax.lax.sort((dst.astype(jnp.int32), iota), num_keys=1,
                              is_stable=False)
    x1 = jax.nn.relu(_gat_conv(x, src, dst, W1, a_src1, a_dst1, b1, n, perm,
                               sdst, _pallas_gather_rows))
    x2 = _gat_conv(x1, src, dst, W2, a_src2, a_dst2, b2, n, perm, sdst,
                   _sc_gather)
    value = (_sc_gather(x2, edge_index[0]) * _sc_gather(x2, edge_index[1])).sum(axis=1)
    E = value.shape[0]
    k_homo = int(E * 0.95)
    k_het = int(E * 0.05)
    _, topk_homo = jax.lax.top_k(value, k_homo)
    _, topk_hetero = jax.lax.top_k(-value, k_het)
    ei_t = edge_index.T
    homo = _sc_gather(ei_t, topk_homo).T
    het = _sc_gather(ei_t, topk_hetero).T
    return homo, het
